# baseline (speedup 1.0000x reference)
import functools

import jax
import jax.numpy as jnp
from jax import lax
from jax.experimental import pallas as pl
from jax.experimental.pallas import tpu as pltpu
from jax.experimental.pallas import tpu_sc as plsc

K = 40
SUB = 2048
BIS = 16
CAP = 512
NW = 32
RPW = 320
L = 16


def _encoder_body(x_ref, w1, b1, w2, b2, w3, b3, wet, web, be,
                  enc_out, a_out, b_out, sqh_out):
    x = x_ref[...]
    h = jax.nn.silu(jnp.dot(x, w1[...], preferred_element_type=jnp.float32) + b1[...])
    h = jax.nn.silu(jnp.dot(h, w2[...], preferred_element_type=jnp.float32) + b2[...])
    e = jnp.dot(h, w3[...], preferred_element_type=jnp.float32) + b3[...]
    enc_out[...] = e
    a_out[...] = jnp.dot(e, wet[...], preferred_element_type=jnp.float32) + be[...]
    b_out[...] = jnp.dot(e, web[...], preferred_element_type=jnp.float32)
    sqh_out[...] = 0.5 * jnp.sum(e * e, axis=1, keepdims=True)


def _score_body(enc_r, et, sqh_t, s_out, t_out):
    s0 = (jnp.dot(enc_r[...], et[...], preferred_element_type=jnp.float32)
          - sqh_t[...])
    s_out[...] = s0
    nsub = min(SUB, s0.shape[1])
    sub = s0[:, :nsub]
    lo = jnp.min(sub, axis=1, keepdims=True)
    hi = jnp.max(sub, axis=1, keepdims=True)

    def body(_, carry):
        lo, hi = carry
        mid = 0.5 * (lo + hi)
        cnt = jnp.sum(jnp.where(sub >= mid, 1.0, 0.0), axis=1, keepdims=True)
        ok = cnt >= K
        return jnp.where(ok, mid, lo), jnp.where(ok, hi, mid)

    lo, hi = lax.fori_loop(0, BIS, body, (lo, hi), unroll=False)
    t_out[...] = lo


def _splat(x, dtype=jnp.int32):
    return jnp.full((L,), x, dtype)


def _desc_key(sv):
    k = plsc.bitcast(sv, jnp.int32)
    ka = jnp.where(k >= 0, k, k ^ jnp.int32(0x7FFFFFFF))
    return jnp.uint32(0x7FFFFFFF) - plsc.bitcast(ka, jnp.uint32)


def _sc_body(s_hbm, that_hbm, a_hbm, b_hbm, out_hbm,
             rowbuf, tbuf, cidx, ckey, hist, chist, selidx, brows,
             arow, frow, sem):
    n = out_hbm.shape[0]
    ncols = s_hbm.shape[1]
    nv_row = ncols // L
    wid = lax.axis_index("s") * 2 + lax.axis_index("c")
    base = wid * RPW
    nrows = jnp.clip(n - base, 0, RPW)
    pltpu.sync_copy(that_hbm.at[pl.ds(base, RPW)], tbuf)
    lane = lax.iota(jnp.int32, L)

    def row_body(r, _):
        row_g = base + r
        with jax.named_scope("rowdma"):
            pltpu.sync_copy(s_hbm.at[row_g], rowbuf)
        t_splat = plsc.load_gather(tbuf, [_splat(r)])

        def cbody(v, cnt_s):
            sv = rowbuf[pl.ds(v * L, L)]
            m = (sv >= t_splat) & (cnt_s <= CAP - L)
            pos = cnt_s + plsc.cumsum(m.astype(jnp.int32)) - 1
            plsc.store_scatter(cidx, [pos], lane + v * L, mask=m)
            return cnt_s + plsc.all_reduce_population_count(m)

        with jax.named_scope("compact"):
            cnt_s = lax.fori_loop(0, nv_row, cbody, _splat(0),
                                  unroll=8 if isinstance(nv_row, int) else None)
            cnt = jnp.max(cnt_s)
        nv = (cnt + L - 1) // L

        def kbody(v, _):
            lv = (lane + v * L) < _splat(cnt)
            sv = plsc.load_gather(rowbuf, [cidx[pl.ds(v * L, L)]], mask=lv)
            kd = jnp.where(lv, _desc_key(sv), jnp.uint32(0xFFFFFFFF))
            ckey[pl.ds(v * L, L)] = kd
            return 0

        with jax.named_scope("keys"):
            lax.fori_loop(0, nv, kbody, 0)

        prefix = jnp.uint32(0)
        need_s = _splat(K)
        scope_radix = jax.named_scope("radix"); scope_radix.__enter__()
        for level in range(4):
            sh = 24 - 8 * level

            def clr(v, _):
                hist[pl.ds(v * L, L)] = jnp.zeros((L,), jnp.int32)
                return 0

            lax.fori_loop(0, 256 // L, clr, 0, unroll=4)

            pref_hi = prefix >> jnp.uint32(sh + 8) if level > 0 else None

            def hbody(v, _):
                kd = ckey[pl.ds(v * L, L)]
                hm = (kd >> jnp.uint32(sh + 8)) == jnp.full((L,), pref_hi) \
                    if level > 0 else jnp.full((L,), True)
                digit = ((kd >> jnp.uint32(sh)) & jnp.uint32(0xFF)).astype(jnp.int32)
                plsc.addupdate_scatter(hist, [digit], jnp.ones((L,), jnp.int32),
                                       mask=hm)
                return 0

            lax.fori_loop(0, nv, hbody, 0)

            def csbody(v, carry_s):
                c = plsc.cumsum(hist[pl.ds(v * L, L)]) + carry_s
                chist[pl.ds(v * L, L)] = c
                return plsc.load_gather(chist, [_splat(v * L + L - 1)])

            lax.fori_loop(0, 256 // L, csbody, _splat(0), unroll=4)

            def fbody(v, acc_s):
                c = chist[pl.ds(v * L, L)]
                return acc_s + plsc.all_reduce_population_count(c < need_s)

            bstar_s = lax.fori_loop(0, 256 // L, fbody, _splat(0), unroll=4)
            bstar = jnp.max(bstar_s)
            cb_vec = plsc.load_gather(chist, [_splat(jnp.maximum(bstar - 1, 0))])
            need_s = need_s - jnp.where(bstar_s == 0, 0, cb_vec)
            prefix = prefix | (bstar.astype(jnp.uint32) << jnp.uint32(sh))

        scope_radix.__exit__(None, None, None)
        t40k = prefix
        t40s = jnp.full((L,), t40k)

        def sbody(v, carry):
            scnt_s, ecnt_s = carry
            kd = ckey[pl.ds(v * L, L)]
            civ = cidx[pl.ds(v * L, L)]
            mlt = kd < t40s
            meq = kd == t40s
            ecs = plsc.cumsum(meq.astype(jnp.int32))
            take = mlt | (meq & ((ecs + ecnt_s) <= need_s))
            pos = scnt_s + plsc.cumsum(take.astype(jnp.int32)) - 1
            plsc.store_scatter(selidx, [pos], civ, mask=take)
            return (scnt_s + plsc.all_reduce_population_count(take),
                    ecnt_s + plsc.all_reduce_population_count(meq))

        with jax.named_scope("select"):
            lax.fori_loop(0, nv, sbody, (_splat(0), _splat(0)))

        nmax = n - 1

        def gbody(k, _):
            sel = jnp.clip(jnp.max(plsc.load_gather(selidx, [_splat(k)])),
                           0, nmax)
            pltpu.async_copy(b_hbm.at[pl.ds(sel * L, L)],
                             brows.at[pl.ds(k * L, L)], sem)
            return 0

        with jax.named_scope("gatherb"):
            lax.fori_loop(0, K, gbody, 0, unroll=8)

        def dbody(k, _):
            pltpu.make_async_copy(b_hbm.at[pl.ds(0, L)],
                                  brows.at[pl.ds(0, L)], sem).wait()
            return 0

        with jax.named_scope("drain"):
            lax.fori_loop(0, K, dbody, 0)
        pltpu.sync_copy(a_hbm.at[row_g], arow)
        av = arow[...]

        def abody(k, acc):
            x = av + brows[pl.ds(k * L, L)]
            return acc + x / (1.0 + jnp.exp(-x))

        with jax.named_scope("agg"):
            acc = lax.fori_loop(0, K, abody, jnp.zeros((L,), jnp.float32), unroll=8)
        frow[...] = acc * (1.0 / K)
        pltpu.sync_copy(frow, out_hbm.at[row_g])
        return 0

    lax.fori_loop(0, nrows, row_body, 0)


def _ffn_body(f_ref, wf1, bf1, wf2, bf2, out_h):
    h = jax.nn.silu(jnp.dot(f_ref[...], wf1[...],
                            preferred_element_type=jnp.float32) + bf1[...])
    out_h[...] = jnp.dot(h, wf2[...], preferred_element_type=jnp.float32) + bf2[...]


def _tc_front(x_pfc, W1, b1, W2, b2, W3, b3, We, be):
    n, d_in = x_pfc.shape
    H = W3.shape[1]
    wet = We[:H] - We[H:]
    web = We[H:]

    r_enc = 1000 if n % 1000 == 0 else n
    enc, a, b, sqh = pl.pallas_call(
        _encoder_body,
        grid=(n // r_enc,),
        in_specs=[
            pl.BlockSpec((r_enc, d_in), lambda i: (i, 0)),
            pl.BlockSpec(W1.shape, lambda i: (0, 0)),
            pl.BlockSpec((1, W1.shape[1]), lambda i: (0, 0)),
            pl.BlockSpec(W2.shape, lambda i: (0, 0)),
            pl.BlockSpec((1, W2.shape[1]), lambda i: (0, 0)),
            pl.BlockSpec(W3.shape, lambda i: (0, 0)),
            pl.BlockSpec((1, W3.shape[1]), lambda i: (0, 0)),
            pl.BlockSpec((H, H), lambda i: (0, 0)),
            pl.BlockSpec((H, H), lambda i: (0, 0)),
            pl.BlockSpec((1, H), lambda i: (0, 0)),
        ],
        out_specs=[
            pl.BlockSpec((r_enc, H), lambda i: (i, 0)),
            pl.BlockSpec((r_enc, H), lambda i: (i, 0)),
            pl.BlockSpec((r_enc, H), lambda i: (i, 0)),
            pl.BlockSpec((r_enc, 1), lambda i: (i, 0)),
        ],
        out_shape=[
            jax.ShapeDtypeStruct((n, H), jnp.float32),
            jax.ShapeDtypeStruct((n, H), jnp.float32),
            jax.ShapeDtypeStruct((n, H), jnp.float32),
            jax.ShapeDtypeStruct((n, 1), jnp.float32),
        ],
    )(x_pfc, W1, b1.reshape(1, -1), W2, b2.reshape(1, -1),
      W3, b3.reshape(1, -1), wet, web, be.reshape(1, -1))

    et = enc.T
    sqh_t = sqh.T

    r_sc = 400 if n % 400 == 0 else n
    s_mat, t_hat = pl.pallas_call(
        _score_body,
        grid=(n // r_sc,),
        in_specs=[
            pl.BlockSpec((r_sc, H), lambda i: (i, 0)),
            pl.BlockSpec((H, n), lambda i: (0, 0)),
            pl.BlockSpec((1, n), lambda i: (0, 0)),
        ],
        out_specs=[
            pl.BlockSpec((r_sc, n), lambda i: (i, 0)),
            pl.BlockSpec((r_sc, 1), lambda i: (i, 0)),
        ],
        out_shape=[
            jax.ShapeDtypeStruct((n, n), jnp.float32),
            jax.ShapeDtypeStruct((n, 1), jnp.float32),
        ],
    )(enc, et, sqh_t)
    return enc, a, b, s_mat, t_hat


def _run_sc(s_mat, t_hat, a, b):
    n, H = a.shape
    that_pad = jnp.pad(t_hat.reshape(-1), (0, NW * RPW - n))
    feats = pl.kernel(
        _sc_body,
        out_type=jax.ShapeDtypeStruct((n, H), jnp.float32),
        mesh=plsc.VectorSubcoreMesh(core_axis_name="c", subcore_axis_name="s",
                                    num_cores=2, num_subcores=16),
        compiler_params=pltpu.CompilerParams(needs_layout_passes=False),
        scratch_types=[
            pltpu.VMEM((n,), jnp.float32),
            pltpu.VMEM((RPW,), jnp.float32),
            pltpu.VMEM((CAP + L,), jnp.int32),
            pltpu.VMEM((CAP + L,), jnp.uint32),
            pltpu.VMEM((256,), jnp.int32),
            pltpu.VMEM((256,), jnp.int32),
            pltpu.VMEM((K + 2 * L,), jnp.int32),
            pltpu.VMEM((K * L,), jnp.float32),
            pltpu.VMEM((H,), jnp.float32),
            pltpu.VMEM((H,), jnp.float32),
            pltpu.SemaphoreType.DMA,
        ],
    )(s_mat, that_pad, a, b.reshape(-1))
    return feats


def _tc_back(feats, x_pfc, Wf1, bf1, Wf2, bf2):
    n, H = feats.shape
    h = pl.pallas_call(
        _ffn_body,
        grid=(1,),
        in_specs=[
            pl.BlockSpec((n, H), lambda i: (0, 0)),
            pl.BlockSpec(Wf1.shape, lambda i: (0, 0)),
            pl.BlockSpec((1, Wf1.shape[1]), lambda i: (0, 0)),
            pl.BlockSpec(Wf2.shape, lambda i: (0, 0)),
            pl.BlockSpec((1, Wf2.shape[1]), lambda i: (0, 0)),
        ],
        out_specs=pl.BlockSpec((n, H), lambda i: (0, 0)),
        out_shape=jax.ShapeDtypeStruct((n, H), jnp.float32),
    )(feats, Wf1, bf1.reshape(1, -1), Wf2, bf2.reshape(1, -1))

    return jnp.concatenate([h, x_pfc], axis=1)


@jax.jit
def kernel(x_pfc, W1, b1, W2, b2, W3, b3, We, be, Wf1, bf1, Wf2, bf2):
    enc, a, b, s_mat, t_hat = _tc_front(x_pfc, W1, b1, W2, b2, W3, b3, We, be)
    feats = _run_sc(s_mat, t_hat, a, b)
    return _tc_back(feats, x_pfc, Wf1, bf1, Wf2, bf2)


# speedup vs baseline: 1.0181x; 1.0181x over previous
import functools

import jax
import jax.numpy as jnp
from jax import lax
from jax.experimental import pallas as pl
from jax.experimental.pallas import tpu as pltpu
from jax.experimental.pallas import tpu_sc as plsc

K = 40
SUB = 2048
BIS = 16
CAP = 512
NW = 32
MIDCAP = 656
RPW = 320
L = 16


def _encoder_body(x_ref, w1, b1, w2, b2, w3, b3, wet, web, be,
                  enc_out, a_out, b_out, sqh_out):
    x = x_ref[...]
    h = jax.nn.silu(jnp.dot(x, w1[...], preferred_element_type=jnp.float32) + b1[...])
    h = jax.nn.silu(jnp.dot(h, w2[...], preferred_element_type=jnp.float32) + b2[...])
    e = jnp.dot(h, w3[...], preferred_element_type=jnp.float32) + b3[...]
    enc_out[...] = e
    a_out[...] = jnp.dot(e, wet[...], preferred_element_type=jnp.float32) + be[...]
    b_out[...] = jnp.dot(e, web[...], preferred_element_type=jnp.float32)
    sqh_out[...] = 0.5 * jnp.sum(e * e, axis=1, keepdims=True)


def _score_body(enc_r, et, sqh_t, s_out, t_out):
    s0 = (jnp.dot(enc_r[...], et[...], preferred_element_type=jnp.float32)
          - sqh_t[...])
    s_out[...] = s0
    nsub = min(SUB, s0.shape[1])
    sub = s0[:, :nsub]
    lo = jnp.min(sub, axis=1, keepdims=True)
    hi = jnp.max(sub, axis=1, keepdims=True)

    def body(_, carry):
        lo, hi = carry
        mid = 0.5 * (lo + hi)
        cnt = jnp.sum(jnp.where(sub >= mid, 1.0, 0.0), axis=1, keepdims=True)
        ok = cnt >= K
        return jnp.where(ok, mid, lo), jnp.where(ok, hi, mid)

    lo, hi = lax.fori_loop(0, BIS, body, (lo, hi), unroll=False)
    t_out[...] = lo


def _splat(x, dtype=jnp.int32):
    return jnp.full((L,), x, dtype)


def _desc_key(sv):
    k = plsc.bitcast(sv, jnp.int32)
    ka = jnp.where(k >= 0, k, k ^ jnp.int32(0x7FFFFFFF))
    return jnp.uint32(0x7FFFFFFF) - plsc.bitcast(ka, jnp.uint32)


def _sc_body(s_hbm, that_hbm, a_hbm, b_hbm, out_hbm,
             rowbuf, tbuf, cidx, ckey, tmp, pcbuf, midbuf, moff, mpc, scr16,
             hist, chist, selidx, brows, arow, frow, sem):
    n = out_hbm.shape[0]
    ncols = s_hbm.shape[1]
    nv_row = ncols // L
    wid = lax.axis_index("s") * 2 + lax.axis_index("c")
    base = wid * RPW
    nrows = jnp.clip(n - base, 0, RPW)
    pltpu.sync_copy(that_hbm.at[pl.ds(base, RPW)], tbuf)
    lane = lax.iota(jnp.int32, L)

    def row_body(r, _):
        row_g = base + r
        pltpu.sync_copy(s_hbm.at[row_g], rowbuf)
        t_splat = plsc.load_gather(tbuf, [_splat(r)])

        def cbody(v, _):
            sv = rowbuf[pl.ds(v * L, L)]
            m = sv >= t_splat
            plsc.store_compressed(tmp.at[pl.ds(v * L, L)], lane + v * L, mask=m)
            pcv = plsc.all_reduce_population_count(m)
            plsc.store_scatter(pcbuf, [_splat(v)], pcv, mask=lane == 0)
            return 0

        lax.fori_loop(0, nv_row, cbody, 0,
                      unroll=8 if isinstance(nv_row, int) else None)
        plsc.store_scatter(pcbuf, [nv_row - 1 + lane], _splat(0),
                           mask=lane >= 1)

        nvb = (nv_row + L - 1) // L

        def bbody(v2, carry):
            total_s, nm_s = carry
            pc = pcbuf[pl.ds(v2 * L, L)]
            nz = pc > 0
            incl = plsc.cumsum(pc)
            offs = total_s + incl - pc
            posm = nm_s + plsc.cumsum(nz.astype(jnp.int32)) - 1
            posm = jnp.minimum(posm, MIDCAP - 1)
            plsc.store_scatter(midbuf, [posm], lane + v2 * L, mask=nz)
            plsc.store_scatter(moff, [posm], offs, mask=nz)
            plsc.store_scatter(mpc, [posm], pc, mask=nz)
            scr16[pl.ds(0, L)] = incl
            tot_last = plsc.load_gather(scr16, [_splat(L - 1)])
            scr16[pl.ds(0, L)] = posm
            nm_last = plsc.load_gather(scr16, [_splat(L - 1)])
            return total_s + tot_last, nm_last + 1

        total_s, nm_s = lax.fori_loop(0, nvb, bbody, (_splat(0), _splat(0)))
        cnt = jnp.minimum(jnp.max(total_s), CAP)
        nv = (cnt + L - 1) // L
        nm = jnp.max(nm_s)

        def mbody(j, _):
            vid = plsc.load_gather(midbuf, [_splat(j)])
            off = plsc.load_gather(moff, [_splat(j)])
            pc = plsc.load_gather(mpc, [_splat(j)])
            vals = plsc.load_gather(tmp, [vid * L + lane], mask=lane < pc)
            pos = jnp.minimum(off + lane, CAP + L - 1)
            plsc.store_scatter(cidx, [pos], vals, mask=lane < pc)
            return 0

        lax.fori_loop(0, nm, mbody, 0)

        def kbody(v, _):
            lv = (lane + v * L) < _splat(cnt)
            sv = plsc.load_gather(rowbuf, [cidx[pl.ds(v * L, L)]], mask=lv)
            kd = jnp.where(lv, _desc_key(sv), jnp.uint32(0xFFFFFFFF))
            ckey[pl.ds(v * L, L)] = kd
            return 0

        lax.fori_loop(0, nv, kbody, 0)

        prefix = jnp.uint32(0)
        need_s = _splat(K)
        for level in range(4):
            sh = 24 - 8 * level

            def clr(v, _):
                hist[pl.ds(v * L, L)] = jnp.zeros((L,), jnp.int32)
                return 0

            lax.fori_loop(0, 256 // L, clr, 0, unroll=4)

            pref_hi = prefix >> jnp.uint32(sh + 8) if level > 0 else None

            def hbody(v, _):
                kd = ckey[pl.ds(v * L, L)]
                hm = (kd >> jnp.uint32(sh + 8)) == jnp.full((L,), pref_hi) \
                    if level > 0 else jnp.full((L,), True)
                digit = ((kd >> jnp.uint32(sh)) & jnp.uint32(0xFF)).astype(jnp.int32)
                plsc.addupdate_scatter(hist, [digit], jnp.ones((L,), jnp.int32),
                                       mask=hm)
                return 0

            lax.fori_loop(0, nv, hbody, 0)

            def csbody(v, carry_s):
                c = plsc.cumsum(hist[pl.ds(v * L, L)]) + carry_s
                chist[pl.ds(v * L, L)] = c
                return plsc.load_gather(chist, [_splat(v * L + L - 1)])

            lax.fori_loop(0, 256 // L, csbody, _splat(0), unroll=4)

            def fbody(v, acc_s):
                c = chist[pl.ds(v * L, L)]
                return acc_s + plsc.all_reduce_population_count(c < need_s)

            bstar_s = lax.fori_loop(0, 256 // L, fbody, _splat(0), unroll=4)
            bstar = jnp.max(bstar_s)
            cb_vec = plsc.load_gather(chist, [_splat(jnp.maximum(bstar - 1, 0))])
            need_s = need_s - jnp.where(bstar_s == 0, 0, cb_vec)
            prefix = prefix | (bstar.astype(jnp.uint32) << jnp.uint32(sh))

        t40k = prefix
        t40s = jnp.full((L,), t40k)

        def sbody(v, carry):
            scnt_s, ecnt_s = carry
            kd = ckey[pl.ds(v * L, L)]
            civ = cidx[pl.ds(v * L, L)]
            mlt = kd < t40s
            meq = kd == t40s
            ecs = plsc.cumsum(meq.astype(jnp.int32))
            take = mlt | (meq & ((ecs + ecnt_s) <= need_s))
            pos = scnt_s + plsc.cumsum(take.astype(jnp.int32)) - 1
            plsc.store_scatter(selidx, [pos], civ, mask=take)
            return (scnt_s + plsc.all_reduce_population_count(take),
                    ecnt_s + plsc.all_reduce_population_count(meq))

        lax.fori_loop(0, nv, sbody, (_splat(0), _splat(0)))

        nmax = n - 1

        def gbody(k, _):
            sel = jnp.clip(jnp.max(plsc.load_gather(selidx, [_splat(k)])),
                           0, nmax)
            pltpu.async_copy(b_hbm.at[pl.ds(sel * L, L)],
                             brows.at[pl.ds(k * L, L)], sem)
            return 0

        lax.fori_loop(0, K, gbody, 0, unroll=8)

        def dbody(k, _):
            pltpu.make_async_copy(b_hbm.at[pl.ds(0, L)],
                                  brows.at[pl.ds(0, L)], sem).wait()
            return 0

        lax.fori_loop(0, K, dbody, 0)
        pltpu.sync_copy(a_hbm.at[row_g], arow)
        av = arow[...]

        def abody(k, acc):
            x = av + brows[pl.ds(k * L, L)]
            return acc + x / (1.0 + jnp.exp(-x))

        acc = lax.fori_loop(0, K, abody, jnp.zeros((L,), jnp.float32), unroll=8)
        frow[...] = acc * (1.0 / K)
        pltpu.sync_copy(frow, out_hbm.at[row_g])
        return 0

    lax.fori_loop(0, nrows, row_body, 0)


def _ffn_body(f_ref, wf1, bf1, wf2, bf2, out_h):
    h = jax.nn.silu(jnp.dot(f_ref[...], wf1[...],
                            preferred_element_type=jnp.float32) + bf1[...])
    out_h[...] = jnp.dot(h, wf2[...], preferred_element_type=jnp.float32) + bf2[...]


def _tc_front(x_pfc, W1, b1, W2, b2, W3, b3, We, be):
    n, d_in = x_pfc.shape
    H = W3.shape[1]
    wet = We[:H] - We[H:]
    web = We[H:]

    r_enc = 1000 if n % 1000 == 0 else n
    enc, a, b, sqh = pl.pallas_call(
        _encoder_body,
        grid=(n // r_enc,),
        in_specs=[
            pl.BlockSpec((r_enc, d_in), lambda i: (i, 0)),
            pl.BlockSpec(W1.shape, lambda i: (0, 0)),
            pl.BlockSpec((1, W1.shape[1]), lambda i: (0, 0)),
            pl.BlockSpec(W2.shape, lambda i: (0, 0)),
            pl.BlockSpec((1, W2.shape[1]), lambda i: (0, 0)),
            pl.BlockSpec(W3.shape, lambda i: (0, 0)),
            pl.BlockSpec((1, W3.shape[1]), lambda i: (0, 0)),
            pl.BlockSpec((H, H), lambda i: (0, 0)),
            pl.BlockSpec((H, H), lambda i: (0, 0)),
            pl.BlockSpec((1, H), lambda i: (0, 0)),
        ],
        out_specs=[
            pl.BlockSpec((r_enc, H), lambda i: (i, 0)),
            pl.BlockSpec((r_enc, H), lambda i: (i, 0)),
            pl.BlockSpec((r_enc, H), lambda i: (i, 0)),
            pl.BlockSpec((r_enc, 1), lambda i: (i, 0)),
        ],
        out_shape=[
            jax.ShapeDtypeStruct((n, H), jnp.float32),
            jax.ShapeDtypeStruct((n, H), jnp.float32),
            jax.ShapeDtypeStruct((n, H), jnp.float32),
            jax.ShapeDtypeStruct((n, 1), jnp.float32),
        ],
    )(x_pfc, W1, b1.reshape(1, -1), W2, b2.reshape(1, -1),
      W3, b3.reshape(1, -1), wet, web, be.reshape(1, -1))

    et = enc.T
    sqh_t = sqh.T

    r_sc = 400 if n % 400 == 0 else n
    s_mat, t_hat = pl.pallas_call(
        _score_body,
        grid=(n // r_sc,),
        in_specs=[
            pl.BlockSpec((r_sc, H), lambda i: (i, 0)),
            pl.BlockSpec((H, n), lambda i: (0, 0)),
            pl.BlockSpec((1, n), lambda i: (0, 0)),
        ],
        out_specs=[
            pl.BlockSpec((r_sc, n), lambda i: (i, 0)),
            pl.BlockSpec((r_sc, 1), lambda i: (i, 0)),
        ],
        out_shape=[
            jax.ShapeDtypeStruct((n, n), jnp.float32),
            jax.ShapeDtypeStruct((n, 1), jnp.float32),
        ],
    )(enc, et, sqh_t)
    return enc, a, b, s_mat, t_hat


def _run_sc(s_mat, t_hat, a, b):
    n, H = a.shape
    that_pad = jnp.pad(t_hat.reshape(-1), (0, NW * RPW - n))
    feats = pl.kernel(
        _sc_body,
        out_type=jax.ShapeDtypeStruct((n, H), jnp.float32),
        mesh=plsc.VectorSubcoreMesh(core_axis_name="c", subcore_axis_name="s",
                                    num_cores=2, num_subcores=16),
        compiler_params=pltpu.CompilerParams(needs_layout_passes=False),
        scratch_types=[
            pltpu.VMEM((n,), jnp.float32),
            pltpu.VMEM((RPW,), jnp.float32),
            pltpu.VMEM((CAP + L,), jnp.int32),
            pltpu.VMEM((CAP + L,), jnp.uint32),
            pltpu.VMEM((n,), jnp.int32),
            pltpu.VMEM((n // L + L,), jnp.int32),
            pltpu.VMEM((MIDCAP,), jnp.int32),
            pltpu.VMEM((MIDCAP,), jnp.int32),
            pltpu.VMEM((MIDCAP,), jnp.int32),
            pltpu.VMEM((L,), jnp.int32),
            pltpu.VMEM((256,), jnp.int32),
            pltpu.VMEM((256,), jnp.int32),
            pltpu.VMEM((K + 2 * L,), jnp.int32),
            pltpu.VMEM((K * L,), jnp.float32),
            pltpu.VMEM((H,), jnp.float32),
            pltpu.VMEM((H,), jnp.float32),
            pltpu.SemaphoreType.DMA,
        ],
    )(s_mat, that_pad, a, b.reshape(-1))
    return feats


def _tc_back(feats, x_pfc, Wf1, bf1, Wf2, bf2):
    n, H = feats.shape
    h = pl.pallas_call(
        _ffn_body,
        grid=(1,),
        in_specs=[
            pl.BlockSpec((n, H), lambda i: (0, 0)),
            pl.BlockSpec(Wf1.shape, lambda i: (0, 0)),
            pl.BlockSpec((1, Wf1.shape[1]), lambda i: (0, 0)),
            pl.BlockSpec(Wf2.shape, lambda i: (0, 0)),
            pl.BlockSpec((1, Wf2.shape[1]), lambda i: (0, 0)),
        ],
        out_specs=pl.BlockSpec((n, H), lambda i: (0, 0)),
        out_shape=jax.ShapeDtypeStruct((n, H), jnp.float32),
    )(feats, Wf1, bf1.reshape(1, -1), Wf2, bf2.reshape(1, -1))

    return jnp.concatenate([h, x_pfc], axis=1)


@jax.jit
def kernel(x_pfc, W1, b1, W2, b2, W3, b3, We, be, Wf1, bf1, Wf2, bf2):
    enc, a, b, s_mat, t_hat = _tc_front(x_pfc, W1, b1, W2, b2, W3, b3, We, be)
    feats = _run_sc(s_mat, t_hat, a, b)
    return _tc_back(feats, x_pfc, Wf1, bf1, Wf2, bf2)


# speedup vs baseline: 1.0182x; 1.0001x over previous
import functools

import jax
import jax.numpy as jnp
from jax import lax
from jax.experimental import pallas as pl
from jax.experimental.pallas import tpu as pltpu
from jax.experimental.pallas import tpu_sc as plsc

K = 40
SUB = 2048
BIS = 16
CAP = 512
NW = 32
MIDCAP = 656
RPW = 320
L = 16


def _encoder_body(x_ref, w1, b1, w2, b2, w3, b3, wet, web, be,
                  enc_out, a_out, b_out, sqh_out):
    x = x_ref[...]
    h = jax.nn.silu(jnp.dot(x, w1[...], preferred_element_type=jnp.float32) + b1[...])
    h = jax.nn.silu(jnp.dot(h, w2[...], preferred_element_type=jnp.float32) + b2[...])
    e = jnp.dot(h, w3[...], preferred_element_type=jnp.float32) + b3[...]
    enc_out[...] = e
    a_out[...] = jnp.dot(e, wet[...], preferred_element_type=jnp.float32) + be[...]
    b_out[...] = jnp.dot(e, web[...], preferred_element_type=jnp.float32)
    sqh_out[...] = 0.5 * jnp.sum(e * e, axis=1, keepdims=True)


def _score_body(enc_r, et, sqh_t, s_out, t_out):
    s0 = (jnp.dot(enc_r[...], et[...], preferred_element_type=jnp.float32)
          - sqh_t[...])
    s_out[...] = s0
    nsub = min(SUB, s0.shape[1])
    sub = s0[:, :nsub]
    lo = jnp.min(sub, axis=1, keepdims=True)
    hi = jnp.max(sub, axis=1, keepdims=True)

    def body(_, carry):
        lo, hi = carry
        mid = 0.5 * (lo + hi)
        cnt = jnp.sum(jnp.where(sub >= mid, 1.0, 0.0), axis=1, keepdims=True)
        ok = cnt >= K
        return jnp.where(ok, mid, lo), jnp.where(ok, hi, mid)

    lo, hi = lax.fori_loop(0, BIS, body, (lo, hi), unroll=False)
    t_out[...] = lo


def _splat(x, dtype=jnp.int32):
    return jnp.full((L,), x, dtype)


def _desc_key(sv):
    k = plsc.bitcast(sv, jnp.int32)
    ka = jnp.where(k >= 0, k, k ^ jnp.int32(0x7FFFFFFF))
    return jnp.uint32(0x7FFFFFFF) - plsc.bitcast(ka, jnp.uint32)


def _sc_body(s_hbm, that_hbm, a_hbm, b_hbm, out_hbm,
             rowbuf, tbuf, cidx, ckey, tmp, pcbuf, midbuf, moff, mpc, scr16,
             hist, chist, selidx, brows, arow, frow, sem):
    n = out_hbm.shape[0]
    ncols = s_hbm.shape[1]
    nv_row = ncols // L
    wid = lax.axis_index("s") * 2 + lax.axis_index("c")
    base = wid * RPW
    nrows = jnp.clip(n - base, 0, RPW)
    pltpu.sync_copy(that_hbm.at[pl.ds(base, RPW)], tbuf)
    lane = lax.iota(jnp.int32, L)

    def row_body(r, _):
        row_g = base + r
        pltpu.sync_copy(s_hbm.at[row_g], rowbuf)
        t_splat = plsc.load_gather(tbuf, [_splat(r)])

        def cbody(v, vidx_s):
            sv = rowbuf[pl.ds(v * L, L)]
            m = sv >= t_splat
            plsc.store_compressed(tmp.at[pl.ds(v * L, L)], lane + v * L, mask=m)
            pcv = plsc.all_reduce_population_count(m)
            plsc.store_scatter(pcbuf, [vidx_s], pcv, mask=lane == 0)
            return vidx_s + 1

        lax.fori_loop(0, nv_row, cbody, _splat(0),
                      unroll=8 if isinstance(nv_row, int) else None)
        plsc.store_scatter(pcbuf, [nv_row - 1 + lane], _splat(0),
                           mask=lane >= 1)

        nvb = (nv_row + L - 1) // L

        def bbody(v2, carry):
            total_s, nm_s = carry
            pc = pcbuf[pl.ds(v2 * L, L)]
            nz = pc > 0
            incl = plsc.cumsum(pc)
            offs = total_s + incl - pc
            posm = nm_s + plsc.cumsum(nz.astype(jnp.int32)) - 1
            posm = jnp.minimum(posm, MIDCAP - 1)
            plsc.store_scatter(midbuf, [posm], lane + v2 * L, mask=nz)
            plsc.store_scatter(moff, [posm], offs, mask=nz)
            plsc.store_scatter(mpc, [posm], pc, mask=nz)
            scr16[pl.ds(0, L)] = incl
            tot_last = plsc.load_gather(scr16, [_splat(L - 1)])
            scr16[pl.ds(0, L)] = posm
            nm_last = plsc.load_gather(scr16, [_splat(L - 1)])
            return total_s + tot_last, nm_last + 1

        total_s, nm_s = lax.fori_loop(0, nvb, bbody, (_splat(0), _splat(0)))
        cnt = jnp.minimum(jnp.max(total_s), CAP)
        nv = (cnt + L - 1) // L
        nm = jnp.max(nm_s)

        def mbody(j, _):
            vid = plsc.load_gather(midbuf, [_splat(j)])
            off = plsc.load_gather(moff, [_splat(j)])
            pc = plsc.load_gather(mpc, [_splat(j)])
            vals = plsc.load_gather(tmp, [vid * L + lane], mask=lane < pc)
            pos = jnp.minimum(off + lane, CAP + L - 1)
            plsc.store_scatter(cidx, [pos], vals, mask=lane < pc)
            return 0

        lax.fori_loop(0, nm, mbody, 0)

        def kbody(v, _):
            lv = (lane + v * L) < _splat(cnt)
            sv = plsc.load_gather(rowbuf, [cidx[pl.ds(v * L, L)]], mask=lv)
            kd = jnp.where(lv, _desc_key(sv), jnp.uint32(0xFFFFFFFF))
            ckey[pl.ds(v * L, L)] = kd
            return 0

        lax.fori_loop(0, nv, kbody, 0)

        prefix = jnp.uint32(0)
        need_s = _splat(K)
        for level in range(4):
            sh = 24 - 8 * level

            def clr(v, _):
                hist[pl.ds(v * L, L)] = jnp.zeros((L,), jnp.int32)
                return 0

            lax.fori_loop(0, 256 // L, clr, 0, unroll=4)

            pref_hi = prefix >> jnp.uint32(sh + 8) if level > 0 else None

            def hbody(v, _):
                kd = ckey[pl.ds(v * L, L)]
                hm = (kd >> jnp.uint32(sh + 8)) == jnp.full((L,), pref_hi) \
                    if level > 0 else jnp.full((L,), True)
                digit = ((kd >> jnp.uint32(sh)) & jnp.uint32(0xFF)).astype(jnp.int32)
                plsc.addupdate_scatter(hist, [digit], jnp.ones((L,), jnp.int32),
                                       mask=hm)
                return 0

            lax.fori_loop(0, nv, hbody, 0)

            def csbody(v, carry_s):
                c = plsc.cumsum(hist[pl.ds(v * L, L)]) + carry_s
                chist[pl.ds(v * L, L)] = c
                return plsc.load_gather(chist, [_splat(v * L + L - 1)])

            lax.fori_loop(0, 256 // L, csbody, _splat(0), unroll=4)

            def fbody(v, acc_s):
                c = chist[pl.ds(v * L, L)]
                return acc_s + plsc.all_reduce_population_count(c < need_s)

            bstar_s = lax.fori_loop(0, 256 // L, fbody, _splat(0), unroll=4)
            bstar = jnp.max(bstar_s)
            cb_vec = plsc.load_gather(chist, [_splat(jnp.maximum(bstar - 1, 0))])
            need_s = need_s - jnp.where(bstar_s == 0, 0, cb_vec)
            prefix = prefix | (bstar.astype(jnp.uint32) << jnp.uint32(sh))

        t40k = prefix
        t40s = jnp.full((L,), t40k)

        def sbody(v, carry):
            scnt_s, ecnt_s = carry
            kd = ckey[pl.ds(v * L, L)]
            civ = cidx[pl.ds(v * L, L)]
            mlt = kd < t40s
            meq = kd == t40s
            ecs = plsc.cumsum(meq.astype(jnp.int32))
            take = mlt | (meq & ((ecs + ecnt_s) <= need_s))
            pos = scnt_s + plsc.cumsum(take.astype(jnp.int32)) - 1
            plsc.store_scatter(selidx, [pos], civ, mask=take)
            return (scnt_s + plsc.all_reduce_population_count(take),
                    ecnt_s + plsc.all_reduce_population_count(meq))

        lax.fori_loop(0, nv, sbody, (_splat(0), _splat(0)))

        nmax = n - 1

        def gbody(k, _):
            sel = jnp.clip(jnp.max(plsc.load_gather(selidx, [_splat(k)])),
                           0, nmax)
            pltpu.async_copy(b_hbm.at[pl.ds(sel * L, L)],
                             brows.at[pl.ds(k * L, L)], sem)
            return 0

        lax.fori_loop(0, K, gbody, 0, unroll=8)

        def dbody(k, _):
            pltpu.make_async_copy(b_hbm.at[pl.ds(0, L)],
                                  brows.at[pl.ds(0, L)], sem).wait()
            return 0

        lax.fori_loop(0, K, dbody, 0)
        pltpu.sync_copy(a_hbm.at[row_g], arow)
        av = arow[...]

        def abody(k, acc):
            x = av + brows[pl.ds(k * L, L)]
            return acc + x / (1.0 + jnp.exp(-x))

        acc = lax.fori_loop(0, K, abody, jnp.zeros((L,), jnp.float32), unroll=8)
        frow[...] = acc * (1.0 / K)
        pltpu.sync_copy(frow, out_hbm.at[row_g])
        return 0

    lax.fori_loop(0, nrows, row_body, 0)


def _ffn_body(f_ref, wf1, bf1, wf2, bf2, out_h):
    h = jax.nn.silu(jnp.dot(f_ref[...], wf1[...],
                            preferred_element_type=jnp.float32) + bf1[...])
    out_h[...] = jnp.dot(h, wf2[...], preferred_element_type=jnp.float32) + bf2[...]


def _tc_front(x_pfc, W1, b1, W2, b2, W3, b3, We, be):
    n, d_in = x_pfc.shape
    H = W3.shape[1]
    wet = We[:H] - We[H:]
    web = We[H:]

    r_enc = 1000 if n % 1000 == 0 else n
    enc, a, b, sqh = pl.pallas_call(
        _encoder_body,
        grid=(n // r_enc,),
        in_specs=[
            pl.BlockSpec((r_enc, d_in), lambda i: (i, 0)),
            pl.BlockSpec(W1.shape, lambda i: (0, 0)),
            pl.BlockSpec((1, W1.shape[1]), lambda i: (0, 0)),
            pl.BlockSpec(W2.shape, lambda i: (0, 0)),
            pl.BlockSpec((1, W2.shape[1]), lambda i: (0, 0)),
            pl.BlockSpec(W3.shape, lambda i: (0, 0)),
            pl.BlockSpec((1, W3.shape[1]), lambda i: (0, 0)),
            pl.BlockSpec((H, H), lambda i: (0, 0)),
            pl.BlockSpec((H, H), lambda i: (0, 0)),
            pl.BlockSpec((1, H), lambda i: (0, 0)),
        ],
        out_specs=[
            pl.BlockSpec((r_enc, H), lambda i: (i, 0)),
            pl.BlockSpec((r_enc, H), lambda i: (i, 0)),
            pl.BlockSpec((r_enc, H), lambda i: (i, 0)),
            pl.BlockSpec((r_enc, 1), lambda i: (i, 0)),
        ],
        out_shape=[
            jax.ShapeDtypeStruct((n, H), jnp.float32),
            jax.ShapeDtypeStruct((n, H), jnp.float32),
            jax.ShapeDtypeStruct((n, H), jnp.float32),
            jax.ShapeDtypeStruct((n, 1), jnp.float32),
        ],
    )(x_pfc, W1, b1.reshape(1, -1), W2, b2.reshape(1, -1),
      W3, b3.reshape(1, -1), wet, web, be.reshape(1, -1))

    et = enc.T
    sqh_t = sqh.T

    r_sc = 400 if n % 400 == 0 else n
    s_mat, t_hat = pl.pallas_call(
        _score_body,
        grid=(n // r_sc,),
        in_specs=[
            pl.BlockSpec((r_sc, H), lambda i: (i, 0)),
            pl.BlockSpec((H, n), lambda i: (0, 0)),
            pl.BlockSpec((1, n), lambda i: (0, 0)),
        ],
        out_specs=[
            pl.BlockSpec((r_sc, n), lambda i: (i, 0)),
            pl.BlockSpec((r_sc, 1), lambda i: (i, 0)),
        ],
        out_shape=[
            jax.ShapeDtypeStruct((n, n), jnp.float32),
            jax.ShapeDtypeStruct((n, 1), jnp.float32),
        ],
    )(enc, et, sqh_t)
    return enc, a, b, s_mat, t_hat


def _run_sc(s_mat, t_hat, a, b):
    n, H = a.shape
    that_pad = jnp.pad(t_hat.reshape(-1), (0, NW * RPW - n))
    feats = pl.kernel(
        _sc_body,
        out_type=jax.ShapeDtypeStruct((n, H), jnp.float32),
        mesh=plsc.VectorSubcoreMesh(core_axis_name="c", subcore_axis_name="s",
                                    num_cores=2, num_subcores=16),
        compiler_params=pltpu.CompilerParams(needs_layout_passes=False),
        scratch_types=[
            pltpu.VMEM((n,), jnp.float32),
            pltpu.VMEM((RPW,), jnp.float32),
            pltpu.VMEM((CAP + L,), jnp.int32),
            pltpu.VMEM((CAP + L,), jnp.uint32),
            pltpu.VMEM((n,), jnp.int32),
            pltpu.VMEM((n // L + L,), jnp.int32),
            pltpu.VMEM((MIDCAP,), jnp.int32),
            pltpu.VMEM((MIDCAP,), jnp.int32),
            pltpu.VMEM((MIDCAP,), jnp.int32),
            pltpu.VMEM((L,), jnp.int32),
            pltpu.VMEM((256,), jnp.int32),
            pltpu.VMEM((256,), jnp.int32),
            pltpu.VMEM((K + 2 * L,), jnp.int32),
            pltpu.VMEM((K * L,), jnp.float32),
            pltpu.VMEM((H,), jnp.float32),
            pltpu.VMEM((H,), jnp.float32),
            pltpu.SemaphoreType.DMA,
        ],
    )(s_mat, that_pad, a, b.reshape(-1))
    return feats


def _tc_back(feats, x_pfc, Wf1, bf1, Wf2, bf2):
    n, H = feats.shape
    h = pl.pallas_call(
        _ffn_body,
        grid=(1,),
        in_specs=[
            pl.BlockSpec((n, H), lambda i: (0, 0)),
            pl.BlockSpec(Wf1.shape, lambda i: (0, 0)),
            pl.BlockSpec((1, Wf1.shape[1]), lambda i: (0, 0)),
            pl.BlockSpec(Wf2.shape, lambda i: (0, 0)),
            pl.BlockSpec((1, Wf2.shape[1]), lambda i: (0, 0)),
        ],
        out_specs=pl.BlockSpec((n, H), lambda i: (0, 0)),
        out_shape=jax.ShapeDtypeStruct((n, H), jnp.float32),
    )(feats, Wf1, bf1.reshape(1, -1), Wf2, bf2.reshape(1, -1))

    return jnp.concatenate([h, x_pfc], axis=1)


@jax.jit
def kernel(x_pfc, W1, b1, W2, b2, W3, b3, We, be, Wf1, bf1, Wf2, bf2):
    enc, a, b, s_mat, t_hat = _tc_front(x_pfc, W1, b1, W2, b2, W3, b3, We, be)
    feats = _run_sc(s_mat, t_hat, a, b)
    return _tc_back(feats, x_pfc, Wf1, bf1, Wf2, bf2)


# speedup vs baseline: 1.0634x; 1.0444x over previous
import functools

import jax
import jax.numpy as jnp
from jax import lax
from jax.experimental import pallas as pl
from jax.experimental.pallas import tpu as pltpu
from jax.experimental.pallas import tpu_sc as plsc

K = 40
SUB = 2048
BIS = 16
CAP = 512
NW = 32
MIDCAP = 656
RPW = 320
L = 16


def _encoder_body(x_ref, w1, b1, w2, b2, w3, b3, wet, web, be,
                  enc_out, a_out, b_out, sqh_out):
    x = x_ref[...]
    h = jax.nn.silu(jnp.dot(x, w1[...], preferred_element_type=jnp.float32) + b1[...])
    h = jax.nn.silu(jnp.dot(h, w2[...], preferred_element_type=jnp.float32) + b2[...])
    e = jnp.dot(h, w3[...], preferred_element_type=jnp.float32) + b3[...]
    enc_out[...] = e
    a_out[...] = jnp.dot(e, wet[...], preferred_element_type=jnp.float32) + be[...]
    b_out[...] = jnp.dot(e, web[...], preferred_element_type=jnp.float32)
    sqh_out[...] = 0.5 * jnp.sum(e * e, axis=1, keepdims=True)


def _score_body(enc_r, et, sqh_t, s_out, t_out):
    s0 = (jnp.dot(enc_r[...], et[...], preferred_element_type=jnp.float32)
          - sqh_t[...])
    s_out[...] = s0
    nsub = min(SUB, s0.shape[1])
    sub = s0[:, :nsub]
    lo = jnp.min(sub, axis=1, keepdims=True)
    hi = jnp.max(sub, axis=1, keepdims=True)

    def body(_, carry):
        lo, hi = carry
        mid = 0.5 * (lo + hi)
        cnt = jnp.sum(jnp.where(sub >= mid, 1.0, 0.0), axis=1, keepdims=True)
        ok = cnt >= K
        return jnp.where(ok, mid, lo), jnp.where(ok, hi, mid)

    lo, hi = lax.fori_loop(0, BIS, body, (lo, hi), unroll=False)
    t_out[...] = lo


def _splat(x, dtype=jnp.int32):
    return jnp.full((L,), x, dtype)


def _desc_key(sv):
    k = plsc.bitcast(sv, jnp.int32)
    ka = jnp.where(k >= 0, k, k ^ jnp.int32(0x7FFFFFFF))
    return jnp.uint32(0x7FFFFFFF) - plsc.bitcast(ka, jnp.uint32)


def _sc_body(s_hbm, that_hbm, a_hbm, b_hbm, out_hbm,
             rowbuf, tbuf, cidx, ckey, tmp, pcbuf, midbuf, moff, mpc, scr16,
             hist, chist, selidx, brows, arow, frow, sem, sem2):
    n = out_hbm.shape[0]
    ncols = s_hbm.shape[1]
    nv_row = ncols // L
    wid = lax.axis_index("s") * 2 + lax.axis_index("c")
    base = wid * RPW
    nrows = jnp.clip(n - base, 0, RPW)
    pltpu.sync_copy(that_hbm.at[pl.ds(base, RPW)], tbuf)
    lane = lax.iota(jnp.int32, L)

    @pl.when(nrows > 0)
    def _():
        pltpu.async_copy(s_hbm.at[base], rowbuf.at[0], sem2)

    def row_body(r, _):
        row_g = base + r
        p = r & 1
        q = 1 - p
        pltpu.make_async_copy(s_hbm.at[base],
                              rowbuf.at[0], sem2).wait()

        @pl.when(r + 1 < nrows)
        def _():
            pltpu.async_copy(s_hbm.at[row_g + 1], rowbuf.at[q], sem2)

        t_splat = plsc.load_gather(tbuf, [_splat(r)])
        p_s = _splat(p)

        def cbody(v, vidx_s):
            sv = rowbuf[p, pl.ds(v * L, L)]
            m = sv >= t_splat
            plsc.store_compressed(tmp.at[pl.ds(v * L, L)], lane + v * L, mask=m)
            pcv = plsc.all_reduce_population_count(m)
            plsc.store_scatter(pcbuf, [vidx_s], pcv, mask=lane == 0)
            return vidx_s + 1

        lax.fori_loop(0, nv_row, cbody, _splat(0),
                      unroll=8 if isinstance(nv_row, int) else None)
        plsc.store_scatter(pcbuf, [nv_row - 1 + lane], _splat(0),
                           mask=lane >= 1)

        nvb = (nv_row + L - 1) // L

        def bbody(v2, carry):
            total_s, nm_s = carry
            pc = pcbuf[pl.ds(v2 * L, L)]
            nz = pc > 0
            incl = plsc.cumsum(pc)
            offs = total_s + incl - pc
            posm = nm_s + plsc.cumsum(nz.astype(jnp.int32)) - 1
            posm = jnp.minimum(posm, MIDCAP - 1)
            plsc.store_scatter(midbuf, [posm], lane + v2 * L, mask=nz)
            plsc.store_scatter(moff, [posm], offs, mask=nz)
            plsc.store_scatter(mpc, [posm], pc, mask=nz)
            scr16[pl.ds(0, L)] = incl
            tot_last = plsc.load_gather(scr16, [_splat(L - 1)])
            scr16[pl.ds(0, L)] = posm
            nm_last = plsc.load_gather(scr16, [_splat(L - 1)])
            return total_s + tot_last, nm_last + 1

        total_s, nm_s = lax.fori_loop(0, nvb, bbody, (_splat(0), _splat(0)))
        cnt = jnp.minimum(jnp.max(total_s), CAP)
        nv = (cnt + L - 1) // L
        nm = jnp.max(nm_s)

        def mbody(j, _):
            vid = plsc.load_gather(midbuf, [_splat(j)])
            off = plsc.load_gather(moff, [_splat(j)])
            pc = plsc.load_gather(mpc, [_splat(j)])
            vals = plsc.load_gather(tmp, [vid * L + lane], mask=lane < pc)
            pos = jnp.minimum(off + lane, CAP + L - 1)
            plsc.store_scatter(cidx, [pos], vals, mask=lane < pc)
            return 0

        lax.fori_loop(0, nm, mbody, 0)

        def kbody(v, _):
            lv = (lane + v * L) < _splat(cnt)
            sv = plsc.load_gather(rowbuf, [p_s, cidx[pl.ds(v * L, L)]],
                                  mask=lv)
            kd = jnp.where(lv, _desc_key(sv), jnp.uint32(0xFFFFFFFF))
            ckey[pl.ds(v * L, L)] = kd
            return 0

        lax.fori_loop(0, nv, kbody, 0)

        prefix = jnp.uint32(0)
        need_s = _splat(K)
        for level in range(4):
            sh = 24 - 8 * level

            def clr(v, _):
                hist[pl.ds(v * L, L)] = jnp.zeros((L,), jnp.int32)
                return 0

            lax.fori_loop(0, 256 // L, clr, 0, unroll=4)

            pref_hi = prefix >> jnp.uint32(sh + 8) if level > 0 else None

            def hbody(v, _):
                kd = ckey[pl.ds(v * L, L)]
                hm = (kd >> jnp.uint32(sh + 8)) == jnp.full((L,), pref_hi) \
                    if level > 0 else jnp.full((L,), True)
                digit = ((kd >> jnp.uint32(sh)) & jnp.uint32(0xFF)).astype(jnp.int32)
                plsc.addupdate_scatter(hist, [digit], jnp.ones((L,), jnp.int32),
                                       mask=hm)
                return 0

            lax.fori_loop(0, nv, hbody, 0)

            def csbody(v, carry_s):
                c = plsc.cumsum(hist[pl.ds(v * L, L)]) + carry_s
                chist[pl.ds(v * L, L)] = c
                return plsc.load_gather(chist, [_splat(v * L + L - 1)])

            lax.fori_loop(0, 256 // L, csbody, _splat(0), unroll=4)

            def fbody(v, acc_s):
                c = chist[pl.ds(v * L, L)]
                return acc_s + plsc.all_reduce_population_count(c < need_s)

            bstar_s = lax.fori_loop(0, 256 // L, fbody, _splat(0), unroll=4)
            bstar = jnp.max(bstar_s)
            cb_vec = plsc.load_gather(chist, [_splat(jnp.maximum(bstar - 1, 0))])
            need_s = need_s - jnp.where(bstar_s == 0, 0, cb_vec)
            prefix = prefix | (bstar.astype(jnp.uint32) << jnp.uint32(sh))

        t40k = prefix
        t40s = jnp.full((L,), t40k)

        def sbody(v, carry):
            scnt_s, ecnt_s = carry
            kd = ckey[pl.ds(v * L, L)]
            civ = cidx[pl.ds(v * L, L)]
            mlt = kd < t40s
            meq = kd == t40s
            ecs = plsc.cumsum(meq.astype(jnp.int32))
            take = mlt | (meq & ((ecs + ecnt_s) <= need_s))
            pos = scnt_s + plsc.cumsum(take.astype(jnp.int32)) - 1
            plsc.store_scatter(selidx, [pos], civ, mask=take)
            return (scnt_s + plsc.all_reduce_population_count(take),
                    ecnt_s + plsc.all_reduce_population_count(meq))

        lax.fori_loop(0, nv, sbody, (_splat(0), _splat(0)))

        nmax = n - 1

        def gbody(k, _):
            sel = jnp.clip(jnp.max(plsc.load_gather(selidx, [_splat(k)])),
                           0, nmax)
            pltpu.async_copy(b_hbm.at[pl.ds(sel * L, L)],
                             brows.at[pl.ds(k * L, L)], sem)
            return 0

        lax.fori_loop(0, K, gbody, 0, unroll=8)

        def dbody(k, _):
            pltpu.make_async_copy(b_hbm.at[pl.ds(0, L)],
                                  brows.at[pl.ds(0, L)], sem).wait()
            return 0

        lax.fori_loop(0, K, dbody, 0)
        pltpu.sync_copy(a_hbm.at[row_g], arow)
        av = arow[...]

        def abody(k, acc):
            x = av + brows[pl.ds(k * L, L)]
            return acc + x / (1.0 + jnp.exp(-x))

        acc = lax.fori_loop(0, K, abody, jnp.zeros((L,), jnp.float32), unroll=8)
        frow[...] = acc * (1.0 / K)
        pltpu.sync_copy(frow, out_hbm.at[row_g])
        return 0

    lax.fori_loop(0, nrows, row_body, 0)


def _ffn_body(f_ref, wf1, bf1, wf2, bf2, out_h):
    h = jax.nn.silu(jnp.dot(f_ref[...], wf1[...],
                            preferred_element_type=jnp.float32) + bf1[...])
    out_h[...] = jnp.dot(h, wf2[...], preferred_element_type=jnp.float32) + bf2[...]


def _tc_front(x_pfc, W1, b1, W2, b2, W3, b3, We, be):
    n, d_in = x_pfc.shape
    H = W3.shape[1]
    wet = We[:H] - We[H:]
    web = We[H:]

    r_enc = 1000 if n % 1000 == 0 else n
    enc, a, b, sqh = pl.pallas_call(
        _encoder_body,
        grid=(n // r_enc,),
        in_specs=[
            pl.BlockSpec((r_enc, d_in), lambda i: (i, 0)),
            pl.BlockSpec(W1.shape, lambda i: (0, 0)),
            pl.BlockSpec((1, W1.shape[1]), lambda i: (0, 0)),
            pl.BlockSpec(W2.shape, lambda i: (0, 0)),
            pl.BlockSpec((1, W2.shape[1]), lambda i: (0, 0)),
            pl.BlockSpec(W3.shape, lambda i: (0, 0)),
            pl.BlockSpec((1, W3.shape[1]), lambda i: (0, 0)),
            pl.BlockSpec((H, H), lambda i: (0, 0)),
            pl.BlockSpec((H, H), lambda i: (0, 0)),
            pl.BlockSpec((1, H), lambda i: (0, 0)),
        ],
        out_specs=[
            pl.BlockSpec((r_enc, H), lambda i: (i, 0)),
            pl.BlockSpec((r_enc, H), lambda i: (i, 0)),
            pl.BlockSpec((r_enc, H), lambda i: (i, 0)),
            pl.BlockSpec((r_enc, 1), lambda i: (i, 0)),
        ],
        out_shape=[
            jax.ShapeDtypeStruct((n, H), jnp.float32),
            jax.ShapeDtypeStruct((n, H), jnp.float32),
            jax.ShapeDtypeStruct((n, H), jnp.float32),
            jax.ShapeDtypeStruct((n, 1), jnp.float32),
        ],
    )(x_pfc, W1, b1.reshape(1, -1), W2, b2.reshape(1, -1),
      W3, b3.reshape(1, -1), wet, web, be.reshape(1, -1))

    et = enc.T
    sqh_t = sqh.T

    r_sc = 400 if n % 400 == 0 else n
    s_mat, t_hat = pl.pallas_call(
        _score_body,
        grid=(n // r_sc,),
        in_specs=[
            pl.BlockSpec((r_sc, H), lambda i: (i, 0)),
            pl.BlockSpec((H, n), lambda i: (0, 0)),
            pl.BlockSpec((1, n), lambda i: (0, 0)),
        ],
        out_specs=[
            pl.BlockSpec((r_sc, n), lambda i: (i, 0)),
            pl.BlockSpec((r_sc, 1), lambda i: (i, 0)),
        ],
        out_shape=[
            jax.ShapeDtypeStruct((n, n), jnp.float32),
            jax.ShapeDtypeStruct((n, 1), jnp.float32),
        ],
    )(enc, et, sqh_t)
    return enc, a, b, s_mat, t_hat


def _run_sc(s_mat, t_hat, a, b):
    n, H = a.shape
    that_pad = jnp.pad(t_hat.reshape(-1), (0, NW * RPW - n))
    feats = pl.kernel(
        _sc_body,
        out_type=jax.ShapeDtypeStruct((n, H), jnp.float32),
        mesh=plsc.VectorSubcoreMesh(core_axis_name="c", subcore_axis_name="s",
                                    num_cores=2, num_subcores=16),
        compiler_params=pltpu.CompilerParams(needs_layout_passes=False),
        scratch_types=[
            pltpu.VMEM((2, n), jnp.float32),
            pltpu.VMEM((RPW,), jnp.float32),
            pltpu.VMEM((CAP + L,), jnp.int32),
            pltpu.VMEM((CAP + L,), jnp.uint32),
            pltpu.VMEM((n,), jnp.int32),
            pltpu.VMEM((n // L + L,), jnp.int32),
            pltpu.VMEM((MIDCAP,), jnp.int32),
            pltpu.VMEM((MIDCAP,), jnp.int32),
            pltpu.VMEM((MIDCAP,), jnp.int32),
            pltpu.VMEM((L,), jnp.int32),
            pltpu.VMEM((256,), jnp.int32),
            pltpu.VMEM((256,), jnp.int32),
            pltpu.VMEM((K + 2 * L,), jnp.int32),
            pltpu.VMEM((K * L,), jnp.float32),
            pltpu.VMEM((H,), jnp.float32),
            pltpu.VMEM((H,), jnp.float32),
            pltpu.SemaphoreType.DMA,
            pltpu.SemaphoreType.DMA,
        ],
    )(s_mat, that_pad, a, b.reshape(-1))
    return feats


def _tc_back(feats, x_pfc, Wf1, bf1, Wf2, bf2):
    n, H = feats.shape
    h = pl.pallas_call(
        _ffn_body,
        grid=(1,),
        in_specs=[
            pl.BlockSpec((n, H), lambda i: (0, 0)),
            pl.BlockSpec(Wf1.shape, lambda i: (0, 0)),
            pl.BlockSpec((1, Wf1.shape[1]), lambda i: (0, 0)),
            pl.BlockSpec(Wf2.shape, lambda i: (0, 0)),
            pl.BlockSpec((1, Wf2.shape[1]), lambda i: (0, 0)),
        ],
        out_specs=pl.BlockSpec((n, H), lambda i: (0, 0)),
        out_shape=jax.ShapeDtypeStruct((n, H), jnp.float32),
    )(feats, Wf1, bf1.reshape(1, -1), Wf2, bf2.reshape(1, -1))

    return jnp.concatenate([h, x_pfc], axis=1)


@jax.jit
def kernel(x_pfc, W1, b1, W2, b2, W3, b3, We, be, Wf1, bf1, Wf2, bf2):
    enc, a, b, s_mat, t_hat = _tc_front(x_pfc, W1, b1, W2, b2, W3, b3, We, be)
    feats = _run_sc(s_mat, t_hat, a, b)
    return _tc_back(feats, x_pfc, Wf1, bf1, Wf2, bf2)
